# baseline (speedup 1.0000x reference)
import functools

import jax
import jax.numpy as jnp
import numpy as np
from jax import lax
from jax.experimental import pallas as pl
from jax.experimental.pallas import tpu as pltpu
from jax.experimental.pallas import tpu_sc as plsc

PATCH = 16
STDDEV = 2.0
KC = 8


_U32 = np.uint32


def _threefry2x32_np(k1, k2, x0, x1):
    def rotl(x, d):
        return (x << _U32(d)) | (x >> _U32(32 - d))

    rots = [(13, 15, 26, 6), (17, 29, 16, 24)]
    ks = [_U32(k1), _U32(k2), _U32(k1) ^ _U32(k2) ^ _U32(0x1BD11BDA)]
    x = [x0 + ks[0], x1 + ks[1]]
    with np.errstate(over="ignore"):
        for i in range(5):
            for r in rots[i % 2]:
                x[0] = x[0] + x[1]
                x[1] = rotl(x[1], r)
                x[1] = x[0] ^ x[1]
            x[0] = x[0] + ks[(i + 1) % 3]
            x[1] = x[1] + ks[(i + 2) % 3] + _U32(i + 1)
    return x


def _random_bits_np(k1, k2, n):
    if jax.config.jax_threefry_partitionable:
        idx = np.arange(n, dtype=np.uint64)
        c1 = (idx >> np.uint64(32)).astype(_U32)
        c2 = (idx & np.uint64(0xFFFFFFFF)).astype(_U32)
        b1, b2 = _threefry2x32_np(k1, k2, c1, c2)
        return b1 ^ b2
    odd = n % 2
    cnt = np.arange(n + odd, dtype=_U32)
    half = len(cnt) // 2
    o1, o2 = _threefry2x32_np(k1, k2, cnt[:half], cnt[half:])
    out = np.concatenate([o1, o2])
    return out[:n] if odd else out


def _erfinv_np(x):
    x = np.asarray(x, np.float64)
    w = -np.log1p(-x * x)
    ws = w - 2.5
    p_small = np.float64(2.81022636e-08)
    for c in (3.43273939e-07, -3.5233877e-06, -4.39150654e-06, 0.00021858087,
              -0.00125372503, -0.00417768164, 0.246640727, 1.50140941):
        p_small = p_small * ws + c
    wl = np.sqrt(np.maximum(w, 1e-30)) - 3.0
    p_large = np.float64(-0.000200214257)
    for c in (0.000100950558, 0.00134934322, -0.00367342844, 0.00573950773,
              -0.0076224613, 0.00943887047, 1.00167406, 2.83297682):
        p_large = p_large * wl + c
    return np.where(w < 5.0, p_small, p_large) * x


def _shift_map_np(B, H, W):
    HP, WP = H // PATCH, W // PATCH
    n = B * HP * WP
    bits = _random_bits_np(0, 42, n)
    fb = (bits >> _U32(9)) | _U32(0x3F800000)
    f = fb.view(np.float32) - np.float32(1.0)
    lo = np.float32(np.nextafter(np.float32(-1.0), np.float32(0.0)))
    hi = np.float32(1.0)
    u = np.maximum(lo, (f * (hi - lo) + lo).astype(np.float32))
    normal = np.sqrt(2.0) * _erfinv_np(u.astype(np.float64))
    m = np.abs(normal * STDDEV)
    return m.astype(np.int32).reshape(B, HP, WP)


@functools.lru_cache(maxsize=None)
def _plan(B, H, W, C, NW):
    s_np = _shift_map_np(B, H, W)
    sv = np.repeat(np.repeat(s_np, H // s_np.shape[1], axis=1), 1, axis=2)
    sv = sv.reshape(-1)
    groups = []
    ids_list = []
    off = 0
    for v in np.unique(sv):
        idx = np.nonzero(sv == v)[0].astype(np.int32)
        n = len(idx)
        npad = -(-n // (NW * KC)) * (NW * KC)
        idx = np.concatenate([idx, np.full(npad - n, idx[-1], np.int32)])
        groups.append((int(v) % C, off, npad))
        ids_list.append(idx)
        off += npad
    return tuple(groups), np.concatenate(ids_list)


def kernel(inputs):
    x = inputs
    B, H, W, C = x.shape
    info = plsc.get_sparse_core_info()
    NC, NS = info.num_cores, info.num_subcores
    NW = NC * NS
    groups, ids_all = _plan(B, H, W, C, NW)
    NCH = B * H * (W // PATCH)
    CL = PATCH * C
    nreg = C // 16

    mesh = plsc.VectorSubcoreMesh(core_axis_name="c", subcore_axis_name="s")

    @functools.partial(
        pl.kernel,
        mesh=mesh,
        out_type=jax.ShapeDtypeStruct((NCH, CL), jnp.float32),
        scratch_types=[
            pltpu.VMEM((KC,), jnp.int32),
            pltpu.VMEM((KC, CL), jnp.float32),
            pltpu.VMEM((KC, CL), jnp.float32),
            pltpu.SemaphoreType.DMA,
            pltpu.SemaphoreType.DMA,
        ],
    )
    def sc_rot(x_ref, ids_ref, out_ref, idsv, inbuf, outbuf, sem_in, sem_out):
        wid = lax.axis_index("s") * NC + lax.axis_index("c")
        for v, goff, npad in groups:
            n_tile = npad // NW
            nb = n_tile // KC
            tbase = goff + wid * n_tile

            def batch(i, _, v=v, tbase=tbase):
                pltpu.sync_copy(ids_ref.at[pl.ds(tbase + i * KC, KC)], idsv)
                pltpu.async_copy(x_ref.at[idsv], inbuf, sem_in).wait()
                if v == 0:
                    pltpu.async_copy(inbuf, out_ref.at[idsv], sem_out).wait()
                else:
                    lane = lax.iota(jnp.int32, 16)
                    ridx = lax.rem(lane + v, 16)
                    dnums = lax.GatherDimensionNumbers(
                        offset_dims=(), collapsed_slice_dims=(0,),
                        start_index_map=(0,))

                    def lrot(vec):
                        return lax.gather(
                            vec, ridx[:, None], dnums, slice_sizes=(1,),
                            mode=lax.GatherScatterMode.PROMISE_IN_BOUNDS)

                    def chunk(r, _):
                        for pp in range(PATCH):
                            base = pp * C
                            for j in range(nreg - 1):
                                outbuf[r, pl.ds(base + 16 * j, 16)] = (
                                    inbuf[r, pl.ds(base + 16 * j + v, 16)]
                                )
                            a = inbuf[r, pl.ds(base + 16 * (nreg - 1), 16)]
                            b = inbuf[r, pl.ds(base, 16)]
                            outbuf[r, pl.ds(base + 16 * (nreg - 1), 16)] = (
                                jnp.where(lane < 16 - v, lrot(a), lrot(b))
                            )
                        return 0

                    lax.fori_loop(0, KC, chunk, 0)
                    pltpu.async_copy(outbuf, out_ref.at[idsv], sem_out).wait()
                return 0

            lax.fori_loop(0, nb, batch, 0)

    out = sc_rot(x.reshape(NCH, CL), jnp.asarray(ids_all))
    return out.reshape(B, H, W, C)


# speedup vs baseline: 1.0925x; 1.0925x over previous
import functools

import jax
import jax.numpy as jnp
import numpy as np
from jax import lax
from jax.experimental import pallas as pl
from jax.experimental.pallas import tpu as pltpu
from jax.experimental.pallas import tpu_sc as plsc

PATCH = 16
STDDEV = 2.0
KC = 16


_U32 = np.uint32


def _threefry2x32_np(k1, k2, x0, x1):
    def rotl(x, d):
        return (x << _U32(d)) | (x >> _U32(32 - d))

    rots = [(13, 15, 26, 6), (17, 29, 16, 24)]
    ks = [_U32(k1), _U32(k2), _U32(k1) ^ _U32(k2) ^ _U32(0x1BD11BDA)]
    x = [x0 + ks[0], x1 + ks[1]]
    with np.errstate(over="ignore"):
        for i in range(5):
            for r in rots[i % 2]:
                x[0] = x[0] + x[1]
                x[1] = rotl(x[1], r)
                x[1] = x[0] ^ x[1]
            x[0] = x[0] + ks[(i + 1) % 3]
            x[1] = x[1] + ks[(i + 2) % 3] + _U32(i + 1)
    return x


def _random_bits_np(k1, k2, n):
    if jax.config.jax_threefry_partitionable:
        idx = np.arange(n, dtype=np.uint64)
        c1 = (idx >> np.uint64(32)).astype(_U32)
        c2 = (idx & np.uint64(0xFFFFFFFF)).astype(_U32)
        b1, b2 = _threefry2x32_np(k1, k2, c1, c2)
        return b1 ^ b2
    odd = n % 2
    cnt = np.arange(n + odd, dtype=_U32)
    half = len(cnt) // 2
    o1, o2 = _threefry2x32_np(k1, k2, cnt[:half], cnt[half:])
    out = np.concatenate([o1, o2])
    return out[:n] if odd else out


def _erfinv_np(x):
    x = np.asarray(x, np.float64)
    w = -np.log1p(-x * x)
    ws = w - 2.5
    p_small = np.float64(2.81022636e-08)
    for c in (3.43273939e-07, -3.5233877e-06, -4.39150654e-06, 0.00021858087,
              -0.00125372503, -0.00417768164, 0.246640727, 1.50140941):
        p_small = p_small * ws + c
    wl = np.sqrt(np.maximum(w, 1e-30)) - 3.0
    p_large = np.float64(-0.000200214257)
    for c in (0.000100950558, 0.00134934322, -0.00367342844, 0.00573950773,
              -0.0076224613, 0.00943887047, 1.00167406, 2.83297682):
        p_large = p_large * wl + c
    return np.where(w < 5.0, p_small, p_large) * x


def _shift_map_np(B, H, W):
    HP, WP = H // PATCH, W // PATCH
    n = B * HP * WP
    bits = _random_bits_np(0, 42, n)
    fb = (bits >> _U32(9)) | _U32(0x3F800000)
    f = fb.view(np.float32) - np.float32(1.0)
    lo = np.float32(np.nextafter(np.float32(-1.0), np.float32(0.0)))
    hi = np.float32(1.0)
    u = np.maximum(lo, (f * (hi - lo) + lo).astype(np.float32))
    normal = np.sqrt(2.0) * _erfinv_np(u.astype(np.float64))
    m = np.abs(normal * STDDEV)
    return m.astype(np.int32).reshape(B, HP, WP)


@functools.lru_cache(maxsize=None)
def _plan(B, H, W, C, NW):
    s_np = _shift_map_np(B, H, W)
    sv = np.repeat(np.repeat(s_np, H // s_np.shape[1], axis=1), 1, axis=2)
    sv = sv.reshape(-1)
    groups = []
    ids_list = []
    off = 0
    for v in np.unique(sv):
        idx = np.nonzero(sv == v)[0].astype(np.int32)
        n = len(idx)
        npad = -(-n // (NW * KC)) * (NW * KC)
        idx = np.concatenate([idx, np.full(npad - n, idx[-1], np.int32)])
        groups.append((int(v) % C, off, npad))
        ids_list.append(idx)
        off += npad
    return tuple(groups), np.concatenate(ids_list)


def kernel(inputs):
    x = inputs
    B, H, W, C = x.shape
    info = plsc.get_sparse_core_info()
    NC, NS = info.num_cores, info.num_subcores
    NW = NC * NS
    groups, ids_all = _plan(B, H, W, C, NW)
    NCH = B * H * (W // PATCH)
    CL = PATCH * C
    nreg = C // 16

    mesh = plsc.VectorSubcoreMesh(core_axis_name="c", subcore_axis_name="s")

    @functools.partial(
        pl.kernel,
        mesh=mesh,
        out_type=jax.ShapeDtypeStruct((NCH, CL), jnp.float32),
        scratch_types=[
            pltpu.VMEM((2, KC), jnp.int32),
            pltpu.VMEM((2, KC, CL), jnp.float32),
            pltpu.VMEM((2, KC, CL), jnp.float32),
            pltpu.SemaphoreType.DMA,
            pltpu.SemaphoreType.DMA,
        ],
    )
    def sc_rot(x_ref, ids_ref, out_ref, idsv, inbuf, outbuf, sem_in, sem_out):
        wid = lax.axis_index("s") * NC + lax.axis_index("c")

        def wait_gather(slot):
            pltpu.make_async_copy(
                x_ref.at[pl.ds(0, KC)], inbuf.at[slot], sem_in).wait()

        def wait_scatter(slot, from_in):
            src = inbuf if from_in else outbuf
            pltpu.make_async_copy(
                src.at[slot], out_ref.at[pl.ds(0, KC)], sem_out).wait()

        for v, goff, npad in groups:
            n_tile = npad // NW
            nb = n_tile // KC
            tbase = goff + wid * n_tile
            lane = lax.iota(jnp.int32, 16)
            ridx = lax.rem(lane + v, 16)
            dnums = lax.GatherDimensionNumbers(
                offset_dims=(), collapsed_slice_dims=(0,),
                start_index_map=(0,))

            def lrot(vec):
                return lax.gather(
                    vec, ridx[:, None], dnums, slice_sizes=(1,),
                    mode=lax.GatherScatterMode.PROMISE_IN_BOUNDS)

            pltpu.sync_copy(ids_ref.at[pl.ds(tbase, KC)], idsv.at[0])
            pltpu.async_copy(x_ref.at[idsv.at[0]], inbuf.at[0], sem_in)

            def batch(i, _, v=v, tbase=tbase, nb=nb):
                slot = lax.rem(i, 2)
                nslot = lax.rem(i + 1, 2)

                @pl.when(i >= 1)
                def _():
                    wait_scatter(nslot, v == 0)

                @pl.when(i + 1 < nb)
                def _():
                    pltpu.sync_copy(
                        ids_ref.at[pl.ds(tbase + (i + 1) * KC, KC)],
                        idsv.at[nslot])
                    pltpu.async_copy(
                        x_ref.at[idsv.at[nslot]], inbuf.at[nslot], sem_in)

                wait_gather(slot)
                if v == 0:
                    pltpu.async_copy(
                        inbuf.at[slot], out_ref.at[idsv.at[slot]], sem_out)
                else:
                    def chunk(r, _):
                        for pp in range(PATCH):
                            base = pp * C
                            for j in range(nreg - 1):
                                outbuf[slot, r, pl.ds(base + 16 * j, 16)] = (
                                    inbuf[slot, r, pl.ds(base + 16 * j + v, 16)]
                                )
                            a = inbuf[slot, r, pl.ds(base + 16 * (nreg - 1), 16)]
                            b = inbuf[slot, r, pl.ds(base, 16)]
                            outbuf[slot, r, pl.ds(base + 16 * (nreg - 1), 16)] = (
                                jnp.where(lane < 16 - v, lrot(a), lrot(b))
                            )
                        return 0

                    lax.fori_loop(0, KC, chunk, 0)
                    pltpu.async_copy(
                        outbuf.at[slot], out_ref.at[idsv.at[slot]], sem_out)
                return 0

            lax.fori_loop(0, nb, batch, 0)
            wait_scatter(lax.rem(nb - 1, 2), v == 0)

    out = sc_rot(x.reshape(NCH, CL), jnp.asarray(ids_all))
    return out.reshape(B, H, W, C)


# speedup vs baseline: 2.2894x; 2.0956x over previous
import functools

import jax
import jax.numpy as jnp
import numpy as np
from jax import lax
from jax.experimental import pallas as pl
from jax.experimental.pallas import tpu as pltpu
from jax.experimental.pallas import tpu_sc as plsc

PATCH = 16
STDDEV = 2.0
KC = 16


_U32 = np.uint32


def _threefry2x32_np(k1, k2, x0, x1):
    def rotl(x, d):
        return (x << _U32(d)) | (x >> _U32(32 - d))

    rots = [(13, 15, 26, 6), (17, 29, 16, 24)]
    ks = [_U32(k1), _U32(k2), _U32(k1) ^ _U32(k2) ^ _U32(0x1BD11BDA)]
    x = [x0 + ks[0], x1 + ks[1]]
    with np.errstate(over="ignore"):
        for i in range(5):
            for r in rots[i % 2]:
                x[0] = x[0] + x[1]
                x[1] = rotl(x[1], r)
                x[1] = x[0] ^ x[1]
            x[0] = x[0] + ks[(i + 1) % 3]
            x[1] = x[1] + ks[(i + 2) % 3] + _U32(i + 1)
    return x


def _random_bits_np(k1, k2, n):
    if jax.config.jax_threefry_partitionable:
        idx = np.arange(n, dtype=np.uint64)
        c1 = (idx >> np.uint64(32)).astype(_U32)
        c2 = (idx & np.uint64(0xFFFFFFFF)).astype(_U32)
        b1, b2 = _threefry2x32_np(k1, k2, c1, c2)
        return b1 ^ b2
    odd = n % 2
    cnt = np.arange(n + odd, dtype=_U32)
    half = len(cnt) // 2
    o1, o2 = _threefry2x32_np(k1, k2, cnt[:half], cnt[half:])
    out = np.concatenate([o1, o2])
    return out[:n] if odd else out


def _erfinv_np(x):
    x = np.asarray(x, np.float64)
    w = -np.log1p(-x * x)
    ws = w - 2.5
    p_small = np.float64(2.81022636e-08)
    for c in (3.43273939e-07, -3.5233877e-06, -4.39150654e-06, 0.00021858087,
              -0.00125372503, -0.00417768164, 0.246640727, 1.50140941):
        p_small = p_small * ws + c
    wl = np.sqrt(np.maximum(w, 1e-30)) - 3.0
    p_large = np.float64(-0.000200214257)
    for c in (0.000100950558, 0.00134934322, -0.00367342844, 0.00573950773,
              -0.0076224613, 0.00943887047, 1.00167406, 2.83297682):
        p_large = p_large * wl + c
    return np.where(w < 5.0, p_small, p_large) * x


def _shift_map_np(B, H, W):
    HP, WP = H // PATCH, W // PATCH
    n = B * HP * WP
    bits = _random_bits_np(0, 42, n)
    fb = (bits >> _U32(9)) | _U32(0x3F800000)
    f = fb.view(np.float32) - np.float32(1.0)
    lo = np.float32(np.nextafter(np.float32(-1.0), np.float32(0.0)))
    hi = np.float32(1.0)
    u = np.maximum(lo, (f * (hi - lo) + lo).astype(np.float32))
    normal = np.sqrt(2.0) * _erfinv_np(u.astype(np.float64))
    m = np.abs(normal * STDDEV)
    return m.astype(np.int32).reshape(B, HP, WP)


@functools.lru_cache(maxsize=None)
def _chunk_shifts(B, H, W):
    s_np = _shift_map_np(B, H, W)
    sv = np.repeat(np.repeat(s_np, PATCH, axis=1), PATCH // PATCH, axis=2)
    sv = np.repeat(s_np, PATCH, axis=1)
    return np.ascontiguousarray(sv.reshape(-1).astype(np.int32))


def kernel(inputs):
    x = inputs
    B, H, W, C = x.shape
    info = plsc.get_sparse_core_info()
    NC, NS = info.num_cores, info.num_subcores
    NW = NC * NS
    shifts = _chunk_shifts(B, H, W)
    assert int(shifts.max()) < 16
    NCH = B * H * (W // PATCH)
    nreg = C // 16
    CPT = NCH // NW
    NB = CPT // KC

    mesh = plsc.VectorSubcoreMesh(core_axis_name="c", subcore_axis_name="s")

    @functools.partial(
        pl.kernel,
        mesh=mesh,
        out_type=jax.ShapeDtypeStruct((NCH, PATCH, C), jnp.float32),
        scratch_types=[
            pltpu.VMEM((CPT,), jnp.int32),
            pltpu.VMEM((2, KC, PATCH, C), jnp.float32),
            pltpu.SemaphoreType.DMA,
            pltpu.SemaphoreType.DMA,
        ],
    )
    def sc_rot(x_ref, vs_ref, out_ref, vsbuf, inbuf, sem_in, sem_out):
        wid = lax.axis_index("s") * NC + lax.axis_index("c")
        tstart = wid * CPT
        lane = lax.iota(jnp.int32, 16)
        dnums = lax.GatherDimensionNumbers(
            offset_dims=(), collapsed_slice_dims=(0,), start_index_map=(0,))

        pltpu.sync_copy(vs_ref.at[pl.ds(tstart, CPT)], vsbuf)

        def wait_gather(slot):
            pltpu.make_async_copy(
                x_ref.at[pl.ds(0, KC)], inbuf.at[slot], sem_in).wait()

        def wait_scatter(slot):
            pltpu.make_async_copy(
                inbuf.at[slot], out_ref.at[pl.ds(0, KC)], sem_out).wait()

        pltpu.async_copy(x_ref.at[pl.ds(tstart, KC)], inbuf.at[0], sem_in)

        def batch(i, _):
            slot = lax.rem(i, 2)
            nslot = lax.rem(i + 1, 2)

            @pl.when(i >= 1)
            def _():
                wait_scatter(nslot)

            @pl.when(i + 1 < NB)
            def _():
                pltpu.async_copy(
                    x_ref.at[pl.ds(tstart + (i + 1) * KC, KC)],
                    inbuf.at[nslot], sem_in)

            wait_gather(slot)
            vv = vsbuf[pl.ds(i * KC, KC)]

            def chunk(r, _):
                vsp = lax.gather(
                    vv, (jnp.zeros((16,), jnp.int32) + r)[:, None], dnums,
                    slice_sizes=(1,),
                    mode=lax.GatherScatterMode.PROMISE_IN_BOUNDS)
                ridx = lax.rem(lane + vsp, 16)

                def lrot(vec):
                    return lax.gather(
                        vec, ridx[:, None], dnums, slice_sizes=(1,),
                        mode=lax.GatherScatterMode.PROMISE_IN_BOUNDS)

                m = lane < 16 - vsp
                for pp in range(PATCH):
                    regs = [inbuf[slot, r, pp, pl.ds(16 * j, 16)]
                            for j in range(nreg)]
                    rots = [lrot(reg) for reg in regs]
                    for j in range(nreg):
                        inbuf[slot, r, pp, pl.ds(16 * j, 16)] = jnp.where(
                            m, rots[j], rots[(j + 1) % nreg])
                return 0

            lax.fori_loop(0, KC, chunk, 0)
            pltpu.async_copy(
                inbuf.at[slot], out_ref.at[pl.ds(tstart + i * KC, KC)],
                sem_out)
            return 0

        lax.fori_loop(0, NB, batch, 0)
        wait_scatter(lax.rem(NB - 1, 2))

    out = sc_rot(x.reshape(NCH, PATCH, C), jnp.asarray(shifts))
    return out.reshape(B, H, W, C)
